# baseline (speedup 1.0000x reference)
import functools

import jax
import jax.numpy as jnp
from jax import lax
from jax.experimental import pallas as pl
from jax.experimental.pallas import tpu as pltpu
from jax.experimental.pallas import tpu_sc as plsc

NUM_WORKERS = 32
CHUNK = 128
SUBS = 16


def _sc_mesh():
    return plsc.VectorSubcoreMesh(core_axis_name="c", subcore_axis_name="s")


def _sc_aggregate(y, idx2, zeros_d):
    NP, D = zeros_d.shape
    NCH = idx2.shape[0] // NUM_WORKERS
    HALF = NCH // 2
    RPS = NP // SUBS
    out_sds = jax.ShapeDtypeStruct((NP, D), jnp.float32)

    @functools.partial(
        pl.kernel,
        out_type=(out_sds, out_sds),
        mesh=_sc_mesh(),
        scratch_types=[
            pltpu.VMEM((2, CHUNK), jnp.int32),
            pltpu.VMEM((2, CHUNK), jnp.int32),
            pltpu.VMEM((CHUNK, D), jnp.float32),
            pltpu.VMEM((CHUNK, D), jnp.float32),
            pltpu.VMEM_SHARED((NP, D), jnp.float32),
            pltpu.SemaphoreType.DMA,
            pltpu.SemaphoreType.DMA,
            pltpu.SemaphoreType.DMA,
            pltpu.SemaphoreType.DMA,
        ],
    )
    def sc_agg(y_hbm, i_hbm, z_hbm, oa_hbm, ob_hbm,
               idx_a, idx_b, rows_a, rows_b, acc, sia, sib, sga, sgb):
        c = lax.axis_index("c")
        s = lax.axis_index("s")
        w = s * 2 + c
        idx = (idx_a, idx_b)
        rows = (rows_a, rows_b)
        sem_i = (sia, sib)
        sem_g = (sga, sgb)
        pltpu.sync_copy(z_hbm.at[pl.ds(s * RPS, RPS)], acc.at[pl.ds(s * RPS, RPS)])
        plsc.subcore_barrier()

        def idx_load(chunk, p):
            pltpu.async_copy(i_hbm.at[w * NCH + chunk], idx[p], sem_i[p])

        def idx_wait(p):
            pltpu.make_async_copy(i_hbm.at[0], idx[p], sem_i[p]).wait()

        def gather(p):
            pltpu.async_copy(y_hbm.at[idx[p].at[0]], rows[p], sem_g[p])

        def gather_wait(p):
            pltpu.make_async_copy(y_hbm.at[idx[p].at[0]], rows[p], sem_g[p]).wait()

        idx_load(0, 0)
        idx_load(1, 1)
        idx_wait(0)
        gather(0)

        @pl.loop(0, HALF)
        def _(blk):
            for p in (0, 1):
                kk = blk * 2 + p
                o = 1 - p
                if p == 0:
                    idx_wait(o)
                    gather(o)
                else:
                    @pl.when(blk < HALF - 1)
                    def _():
                        idx_wait(o)
                        gather(o)
                gather_wait(p)
                pltpu.sync_copy(rows[p], acc.at[idx[p].at[1]], add=True)

                @pl.when(kk + 2 < NCH)
                def _():
                    idx_load(kk + 2, p)

        plsc.subcore_barrier()

        @pl.when(c == 0)
        def _():
            pltpu.sync_copy(acc.at[pl.ds(s * RPS, RPS)], oa_hbm.at[pl.ds(s * RPS, RPS)])

        @pl.when(c == 1)
        def _():
            pltpu.sync_copy(acc.at[pl.ds(s * RPS, RPS)], ob_hbm.at[pl.ds(s * RPS, RPS)])

    return sc_agg(y, idx2, zeros_d)


_DOT_KW = dict(preferred_element_type=jnp.float32, precision=lax.Precision.HIGHEST)


def _dis_block(da_ref, db_ref):
    return lax.rsqrt(da_ref[:, :1] + db_ref[:, :1] + 1.0)


def _tc_scale_matmul(xp, W, dpa, dpb):
    NP, D = xp.shape
    R = 512

    def body(x_ref, w_ref, da_ref, db_ref, o_ref):
        dis = _dis_block(da_ref, db_ref)
        o_ref[...] = jnp.dot(x_ref[...], w_ref[...], **_DOT_KW) * dis

    return pl.pallas_call(
        body,
        grid=(NP // R,),
        in_specs=[
            pl.BlockSpec((R, D), lambda i: (i, 0)),
            pl.BlockSpec((D, D), lambda i: (0, 0)),
            pl.BlockSpec((R, D), lambda i: (i, 0)),
            pl.BlockSpec((R, D), lambda i: (i, 0)),
        ],
        out_specs=pl.BlockSpec((R, D), lambda i: (i, 0)),
        out_shape=jax.ShapeDtypeStruct((NP, D), jnp.float32),
    )(xp, W, dpa, dpb)


def _tc_layer(pa, pb, y, dpa, dpb, b, W):
    NP, D = y.shape
    R = 512

    def body(pa_ref, pb_ref, y_ref, da_ref, db_ref, b_ref, w_ref, o_ref):
        dis = _dis_block(da_ref, db_ref)
        h = jnp.tanh(dis * (pa_ref[...] + pb_ref[...] + y_ref[...]) + b_ref[...])
        o_ref[...] = jnp.dot(h, w_ref[...], **_DOT_KW) * dis

    return pl.pallas_call(
        body,
        grid=(NP // R,),
        in_specs=[
            pl.BlockSpec((R, D), lambda i: (i, 0)),
            pl.BlockSpec((R, D), lambda i: (i, 0)),
            pl.BlockSpec((R, D), lambda i: (i, 0)),
            pl.BlockSpec((R, D), lambda i: (i, 0)),
            pl.BlockSpec((R, D), lambda i: (i, 0)),
            pl.BlockSpec((1, D), lambda i: (0, 0)),
            pl.BlockSpec((D, D), lambda i: (0, 0)),
        ],
        out_specs=pl.BlockSpec((R, D), lambda i: (i, 0)),
        out_shape=jax.ShapeDtypeStruct((NP, D), jnp.float32),
    )(pa, pb, y, dpa, dpb, b, W)


def _tc_head(pa, pb, y, dpa, dpb, b, W3p, b3p):
    NP, D = y.shape
    R = 512

    def body(pa_ref, pb_ref, y_ref, da_ref, db_ref, b_ref, w_ref, b3_ref,
             h_ref, lg_ref):
        dis = _dis_block(da_ref, db_ref)
        h = jnp.tanh(dis * (pa_ref[...] + pb_ref[...] + y_ref[...]) + b_ref[...])
        h_ref[...] = h
        lg_ref[...] = jnp.dot(h, w_ref[...], **_DOT_KW) + b3_ref[...]

    return pl.pallas_call(
        body,
        grid=(NP // R,),
        in_specs=[
            pl.BlockSpec((R, D), lambda i: (i, 0)),
            pl.BlockSpec((R, D), lambda i: (i, 0)),
            pl.BlockSpec((R, D), lambda i: (i, 0)),
            pl.BlockSpec((R, D), lambda i: (i, 0)),
            pl.BlockSpec((R, D), lambda i: (i, 0)),
            pl.BlockSpec((1, D), lambda i: (0, 0)),
            pl.BlockSpec((D, D), lambda i: (0, 0)),
            pl.BlockSpec((1, D), lambda i: (0, 0)),
        ],
        out_specs=[
            pl.BlockSpec((R, D), lambda i: (i, 0)),
            pl.BlockSpec((R, D), lambda i: (i, 0)),
        ],
        out_shape=[
            jax.ShapeDtypeStruct((NP, D), jnp.float32),
            jax.ShapeDtypeStruct((NP, D), jnp.float32),
        ],
    )(pa, pb, y, dpa, dpb, b, W3p, b3p)


def kernel(x, edge_index, W1, b1, W2, b2, W3, b3):
    N, D = x.shape
    E = edge_index.shape[1]
    NP = -(-N // 512) * 512
    NCH = -(-E // (NUM_WORKERS * CHUNK))
    NCH += NCH % 2
    EPAD = NCH * NUM_WORKERS * CHUNK
    DOUT = W3.shape[1]

    src = edge_index[0].astype(jnp.int32)
    dst = edge_index[1].astype(jnp.int32)
    pad = jnp.full((EPAD - E,), N, jnp.int32)
    src2 = jnp.concatenate([src, pad]).reshape(EPAD // CHUNK, CHUNK)
    dst2 = jnp.concatenate([dst, pad]).reshape(EPAD // CHUNK, CHUNK)
    idx2 = jnp.stack([src2, dst2], axis=1)
    didx2 = jnp.stack([jnp.zeros_like(dst2), dst2], axis=1)
    xp = jnp.pad(x, ((0, NP - N), (0, 0)))
    zeros_d = jnp.zeros((NP, D), jnp.float32)
    W3p = jnp.pad(W3, ((0, 0), (0, D - DOUT)))
    b3p = jnp.pad(b3, (0, D - DOUT)).reshape(1, D)
    b1r = b1.reshape(1, D)
    b2r = b2.reshape(1, D)

    ones_tab = jnp.ones((NP, D), jnp.float32)
    dpa, dpb = _sc_aggregate(ones_tab, didx2, zeros_d)
    y1 = _tc_scale_matmul(xp, W1, dpa, dpb)
    p1a, p1b = _sc_aggregate(y1, idx2, zeros_d)
    y2 = _tc_layer(p1a, p1b, y1, dpa, dpb, b1r, W2)
    p2a, p2b = _sc_aggregate(y2, idx2, zeros_d)
    h2f, lgf = _tc_head(p2a, p2b, y2, dpa, dpb, b2r, W3p, b3p)
    return h2f[:N], lgf[:N, :DOUT]
